# baseline (speedup 1.0000x reference)
import functools

import jax
import jax.numpy as jnp
from jax import lax
from jax.experimental import pallas as pl
from jax.experimental.pallas import tpu as pltpu
from jax.experimental.pallas import tpu_sc as plsc

_RADIUS = 3.0
_NCELL = 16
_NSTEPS = 50
_NCORES = 2
_NSUB = 16
_NW = _NCORES * _NSUB
_LANES = 16
_U = 16
_STEP_UNROLL = 10
_BLK = _U * _LANES


def _prep_tables(theta, ba, bb, time):
    c = theta.shape[0]

    def body(time_ref, theta_ref, ba_ref, bb_ref, a_ref, b_ref):
        dt = time_ref[0] / jnp.float32(_NSTEPS)
        dn = (((1,), (1,)), ((), ()))
        a = lax.dot_general(ba_ref[...], theta_ref[...], dn,
                            preferred_element_type=jnp.float32)
        b = lax.dot_general(bb_ref[...], theta_ref[...], dn,
                            preferred_element_type=jnp.float32)
        a_ref[...] = jnp.float32(1.0) + dt * a
        b_ref[...] = (jnp.float32(16.0) * dt) * b

    return pl.pallas_call(
        body,
        in_specs=[
            pl.BlockSpec(memory_space=pltpu.SMEM),
            pl.BlockSpec(memory_space=pltpu.VMEM),
            pl.BlockSpec(memory_space=pltpu.VMEM),
            pl.BlockSpec(memory_space=pltpu.VMEM),
        ],
        out_specs=[
            pl.BlockSpec(memory_space=pltpu.VMEM),
            pl.BlockSpec(memory_space=pltpu.VMEM),
        ],
        out_shape=[
            jax.ShapeDtypeStruct((_NCELL, c), jnp.float32),
            jax.ShapeDtypeStruct((_NCELL, c), jnp.float32),
        ],
    )(time, theta, ba, bb)


def _sc_transform(xflat, a1flat, b16flat, nchan):
    n_elem = xflat.shape[0]
    nblk = n_elem // _BLK
    assert nblk * _BLK == n_elem
    base_blocks = nblk // _NW
    extra = nblk - base_blocks * _NW
    buf_words = (base_blocks + (1 if extra else 0)) * _BLK

    mesh = plsc.VectorSubcoreMesh(
        core_axis_name="c", subcore_axis_name="s",
        num_cores=_NCORES, num_subcores=_NSUB)

    @functools.partial(
        pl.kernel,
        mesh=mesh,
        compiler_params=pltpu.CompilerParams(needs_layout_passes=False),
        out_type=jax.ShapeDtypeStruct((n_elem,), jnp.float32),
        scratch_types=[
            pltpu.VMEM((buf_words,), jnp.float32),
            pltpu.VMEM((buf_words,), jnp.float32),
            pltpu.VMEM((nchan * _NCELL,), jnp.float32),
            pltpu.VMEM((nchan * _NCELL,), jnp.float32),
        ],
    )
    def run(x_hbm, a_hbm, b_hbm, out_hbm, xin, xout, atab, btab):
        wid = lax.axis_index("s") * _NCORES + lax.axis_index("c")
        is_big = wid < extra
        myblocks = base_blocks + jnp.where(is_big, 1, 0)
        start = wid * base_blocks + jnp.minimum(wid, extra)
        base = start * _BLK

        @pl.when(is_big)
        def _():
            pltpu.sync_copy(x_hbm.at[pl.ds(base, buf_words)], xin)

        @pl.when(jnp.logical_not(is_big))
        def _():
            pltpu.sync_copy(
                x_hbm.at[pl.ds(base, base_blocks * _BLK)],
                xin.at[pl.ds(0, base_blocks * _BLK)])

        pltpu.sync_copy(a_hbm, atab)
        pltpu.sync_copy(b_hbm, btab)

        iota = lax.iota(jnp.int32, _LANES)
        ib8 = [iota + (16 * j % nchan) for j in range(8)]

        def group(g, _):
            offs = g * _BLK
            z = []
            for u in range(_U):
                xs = (xin[pl.ds(offs + u * _LANES, _LANES)]
                      + _RADIUS) * jnp.float32(1.0 / (2.0 * _RADIUS))
                z.append(xs * jnp.float32(_NCELL))

            def step(i, zs):
                zs = list(zs)
                for _ in range(_STEP_UNROLL):
                    out = []
                    for u in range(_U):
                        zu = zs[u]
                        cell = jnp.minimum(
                            jnp.maximum(zu, jnp.float32(0.0)),
                            jnp.float32(_NCELL - 1)).astype(jnp.int32)
                        idx = cell * nchan + ib8[u % 8]
                        ac = plsc.load_gather(atab, [idx])
                        bc = plsc.load_gather(btab, [idx])
                        out.append(ac * zu + bc)
                    zs = out
                return tuple(zs)

            zf = lax.fori_loop(0, _NSTEPS // _STEP_UNROLL, step, tuple(z))

            scale = jnp.float32(2.0 * _RADIUS / _NCELL)
            for u in range(_U):
                xv = xin[pl.ds(offs + u * _LANES, _LANES)]
                xs = (xv + _RADIUS) * jnp.float32(1.0 / (2.0 * _RADIUS))
                msk = jnp.logical_or(xs >= 1.0, xs <= 0.0)
                res = zf[u] * scale - jnp.float32(_RADIUS)
                xout[pl.ds(offs + u * _LANES, _LANES)] = jnp.where(msk, xv, res)
            return 0

        lax.fori_loop(0, myblocks, group, 0)

        @pl.when(is_big)
        def _():
            pltpu.sync_copy(xout, out_hbm.at[pl.ds(base, buf_words)])

        @pl.when(jnp.logical_not(is_big))
        def _():
            pltpu.sync_copy(
                xout.at[pl.ds(0, base_blocks * _BLK)],
                out_hbm.at[pl.ds(base, base_blocks * _BLK)])

    return run(xflat, a1flat, b16flat)


def kernel(x, edge_index, edge_attr, batch, time, theta, B):
    n, nchan = x.shape
    ba = B[0::2, :]
    bb = B[1::2, :]
    a1, b16 = _prep_tables(theta, ba, bb, time)
    yflat = _sc_transform(x.reshape(-1), a1.reshape(-1), b16.reshape(-1), nchan)
    return (yflat.reshape(n, nchan), theta)


# speedup vs baseline: 1.0074x; 1.0074x over previous
import functools

import jax
import jax.numpy as jnp
from jax import lax
from jax.experimental import pallas as pl
from jax.experimental.pallas import tpu as pltpu
from jax.experimental.pallas import tpu_sc as plsc

_RADIUS = 3.0
_NCELL = 16
_NSTEPS = 50
_NCORES = 2
_NSUB = 16
_NW = _NCORES * _NSUB
_LANES = 16
_U = 16
_STEP_UNROLL = 5
_BLK = _U * _LANES


def _prep_tables(theta, ba, bb, time):
    c = theta.shape[0]

    def body(time_ref, theta_ref, ba_ref, bb_ref, a_ref, b_ref):
        dt = time_ref[0] / jnp.float32(_NSTEPS)
        dn = (((1,), (1,)), ((), ()))
        a = lax.dot_general(ba_ref[...], theta_ref[...], dn,
                            preferred_element_type=jnp.float32)
        b = lax.dot_general(bb_ref[...], theta_ref[...], dn,
                            preferred_element_type=jnp.float32)
        a_ref[...] = jnp.float32(1.0) + dt * a
        b_ref[...] = (jnp.float32(16.0) * dt) * b

    return pl.pallas_call(
        body,
        in_specs=[
            pl.BlockSpec(memory_space=pltpu.SMEM),
            pl.BlockSpec(memory_space=pltpu.VMEM),
            pl.BlockSpec(memory_space=pltpu.VMEM),
            pl.BlockSpec(memory_space=pltpu.VMEM),
        ],
        out_specs=[
            pl.BlockSpec(memory_space=pltpu.VMEM),
            pl.BlockSpec(memory_space=pltpu.VMEM),
        ],
        out_shape=[
            jax.ShapeDtypeStruct((_NCELL, c), jnp.float32),
            jax.ShapeDtypeStruct((_NCELL, c), jnp.float32),
        ],
    )(time, theta, ba, bb)


def _sc_transform(xflat, a1flat, b16flat, nchan):
    n_elem = xflat.shape[0]
    nblk = n_elem // _BLK
    assert nblk * _BLK == n_elem
    base_blocks = nblk // _NW
    extra = nblk - base_blocks * _NW
    buf_words = (base_blocks + (1 if extra else 0)) * _BLK

    mesh = plsc.VectorSubcoreMesh(
        core_axis_name="c", subcore_axis_name="s",
        num_cores=_NCORES, num_subcores=_NSUB)

    @functools.partial(
        pl.kernel,
        mesh=mesh,
        compiler_params=pltpu.CompilerParams(needs_layout_passes=False),
        out_type=jax.ShapeDtypeStruct((n_elem,), jnp.float32),
        scratch_types=[
            pltpu.VMEM((buf_words,), jnp.float32),
            pltpu.VMEM((buf_words,), jnp.float32),
            pltpu.VMEM((nchan * _NCELL,), jnp.float32),
            pltpu.VMEM((nchan * _NCELL,), jnp.float32),
        ],
    )
    def run(x_hbm, a_hbm, b_hbm, out_hbm, xin, xout, atab, btab):
        wid = lax.axis_index("s") * _NCORES + lax.axis_index("c")
        is_big = wid < extra
        myblocks = base_blocks + jnp.where(is_big, 1, 0)
        start = wid * base_blocks + jnp.minimum(wid, extra)
        base = start * _BLK

        @pl.when(is_big)
        def _():
            pltpu.sync_copy(x_hbm.at[pl.ds(base, buf_words)], xin)

        @pl.when(jnp.logical_not(is_big))
        def _():
            pltpu.sync_copy(
                x_hbm.at[pl.ds(base, base_blocks * _BLK)],
                xin.at[pl.ds(0, base_blocks * _BLK)])

        pltpu.sync_copy(a_hbm, atab)
        pltpu.sync_copy(b_hbm, btab)

        iota = lax.iota(jnp.int32, _LANES)
        ib8 = [iota + (16 * j % nchan) for j in range(8)]

        @plsc.parallel_loop(0, myblocks)
        def group(g):
            offs = g * _BLK
            z = []
            for u in range(_U):
                xs = (xin[pl.ds(offs + u * _LANES, _LANES)]
                      + _RADIUS) * jnp.float32(1.0 / (2.0 * _RADIUS))
                z.append(xs * jnp.float32(_NCELL))

            def step(i, zs):
                zs = list(zs)
                for _ in range(_STEP_UNROLL):
                    out = []
                    for u in range(_U):
                        zu = zs[u]
                        cell = jnp.minimum(
                            jnp.maximum(zu, jnp.float32(0.0)),
                            jnp.float32(_NCELL - 1)).astype(jnp.int32)
                        idx = cell * nchan + ib8[u % 8]
                        ac = plsc.load_gather(atab, [idx])
                        bc = plsc.load_gather(btab, [idx])
                        out.append(ac * zu + bc)
                    zs = out
                return tuple(zs)

            zf = lax.fori_loop(0, _NSTEPS // _STEP_UNROLL, step, tuple(z))

            scale = jnp.float32(2.0 * _RADIUS / _NCELL)
            for u in range(_U):
                xv = xin[pl.ds(offs + u * _LANES, _LANES)]
                xs = (xv + _RADIUS) * jnp.float32(1.0 / (2.0 * _RADIUS))
                msk = jnp.logical_or(xs >= 1.0, xs <= 0.0)
                res = zf[u] * scale - jnp.float32(_RADIUS)
                xout[pl.ds(offs + u * _LANES, _LANES)] = jnp.where(msk, xv, res)

        @pl.when(is_big)
        def _():
            pltpu.sync_copy(xout, out_hbm.at[pl.ds(base, buf_words)])

        @pl.when(jnp.logical_not(is_big))
        def _():
            pltpu.sync_copy(
                xout.at[pl.ds(0, base_blocks * _BLK)],
                out_hbm.at[pl.ds(base, base_blocks * _BLK)])

    return run(xflat, a1flat, b16flat)


def kernel(x, edge_index, edge_attr, batch, time, theta, B):
    n, nchan = x.shape
    ba = B[0::2, :]
    bb = B[1::2, :]
    a1, b16 = _prep_tables(theta, ba, bb, time)
    yflat = _sc_transform(x.reshape(-1), a1.reshape(-1), b16.reshape(-1), nchan)
    return (yflat.reshape(n, nchan), theta)


# speedup vs baseline: 1.3022x; 1.2926x over previous
import functools

import jax
import jax.numpy as jnp
from jax import lax
from jax.experimental import pallas as pl
from jax.experimental.pallas import tpu as pltpu
from jax.experimental.pallas import tpu_sc as plsc

_RADIUS = 3.0
_NCELL = 16
_NSTEPS = 50
_NCORES = 2
_NSUB = 16
_NW = _NCORES * _NSUB
_LANES = 16
_U = 16
_STEP_UNROLL = 5
_BLK = _U * _LANES
_TC_ROWS = 2880
_TC_BR = 320


def _prep_tables(theta, ba, bb, time):
    c = theta.shape[0]

    def body(time_ref, theta_ref, ba_ref, bb_ref, a_ref, b_ref, s_ref):
        dt = time_ref[0] / jnp.float32(_NSTEPS)
        dn = (((1,), (1,)), ((), ()))
        a = lax.dot_general(ba_ref[...], theta_ref[...], dn,
                            preferred_element_type=jnp.float32)
        b = lax.dot_general(bb_ref[...], theta_ref[...], dn,
                            preferred_element_type=jnp.float32)
        a_ref[...] = jnp.float32(1.0) + dt * a
        b_ref[...] = (jnp.float32(16.0) * dt) * b
        shifted = jnp.concatenate(
            [jnp.zeros((1, c), jnp.float32), a[:_NCELL - 1, :]], axis=0)
        rowid = lax.broadcasted_iota(jnp.int32, (_NCELL, c), 0)
        s_ref[...] = dt * (a - shifted) + jnp.where(
            rowid == 0, jnp.float32(1.0), jnp.float32(0.0))

    return pl.pallas_call(
        body,
        in_specs=[
            pl.BlockSpec(memory_space=pltpu.SMEM),
            pl.BlockSpec(memory_space=pltpu.VMEM),
            pl.BlockSpec(memory_space=pltpu.VMEM),
            pl.BlockSpec(memory_space=pltpu.VMEM),
        ],
        out_specs=[
            pl.BlockSpec(memory_space=pltpu.VMEM),
            pl.BlockSpec(memory_space=pltpu.VMEM),
            pl.BlockSpec(memory_space=pltpu.VMEM),
        ],
        out_shape=[
            jax.ShapeDtypeStruct((_NCELL, c), jnp.float32),
            jax.ShapeDtypeStruct((_NCELL, c), jnp.float32),
            jax.ShapeDtypeStruct((_NCELL, c), jnp.float32),
        ],
    )(time, theta, ba, bb)


def _tc_transform(x2d, s0e):
    r, c = x2d.shape
    br = _TC_BR
    assert r % br == 0

    def body(s_ref, x_ref, o_ref):
        coef = s_ref[...]
        xv = x_ref[...]
        xs = (xv + _RADIUS) * jnp.float32(1.0 / (2.0 * _RADIUS))
        z0 = xs * jnp.float32(_NCELL)
        s0 = coef[0:1, :]

        def step(i, z):
            parts = [z * s0, None, None, None]
            for k in range(1, _NCELL):
                t = coef[k:k + 1, :] * jnp.maximum(
                    z - jnp.float32(k), jnp.float32(0.0))
                j = k % 4
                parts[j] = t if parts[j] is None else parts[j] + t
            return (parts[0] + parts[1]) + (parts[2] + parts[3])

        zf = lax.fori_loop(0, _NSTEPS, step, z0)
        res = zf * jnp.float32(2.0 * _RADIUS / _NCELL) - jnp.float32(_RADIUS)
        msk = jnp.logical_or(xs >= 1.0, xs <= 0.0)
        o_ref[...] = jnp.where(msk, xv, res)

    return pl.pallas_call(
        body,
        grid=(r // br,),
        in_specs=[
            pl.BlockSpec((_NCELL, c), lambda i: (0, 0)),
            pl.BlockSpec((br, c), lambda i: (i, 0)),
        ],
        out_specs=pl.BlockSpec((br, c), lambda i: (i, 0)),
        out_shape=jax.ShapeDtypeStruct((r, c), jnp.float32),
    )(s0e, x2d)


def _sc_transform(xflat, a1flat, b16flat, nchan):
    n_elem = xflat.shape[0]
    nblk = n_elem // _BLK
    assert nblk * _BLK == n_elem
    base_blocks = nblk // _NW
    extra = nblk - base_blocks * _NW
    buf_words = (base_blocks + (1 if extra else 0)) * _BLK

    mesh = plsc.VectorSubcoreMesh(
        core_axis_name="c", subcore_axis_name="s",
        num_cores=_NCORES, num_subcores=_NSUB)

    @functools.partial(
        pl.kernel,
        mesh=mesh,
        compiler_params=pltpu.CompilerParams(needs_layout_passes=False),
        out_type=jax.ShapeDtypeStruct((n_elem,), jnp.float32),
        scratch_types=[
            pltpu.VMEM((buf_words,), jnp.float32),
            pltpu.VMEM((buf_words,), jnp.float32),
            pltpu.VMEM((nchan * _NCELL,), jnp.float32),
            pltpu.VMEM((nchan * _NCELL,), jnp.float32),
        ],
    )
    def run(x_hbm, a_hbm, b_hbm, out_hbm, xin, xout, atab, btab):
        wid = lax.axis_index("s") * _NCORES + lax.axis_index("c")
        is_big = wid < extra
        myblocks = base_blocks + jnp.where(is_big, 1, 0)
        start = wid * base_blocks + jnp.minimum(wid, extra)
        base = start * _BLK

        @pl.when(is_big)
        def _():
            pltpu.sync_copy(x_hbm.at[pl.ds(base, buf_words)], xin)

        @pl.when(jnp.logical_not(is_big))
        def _():
            pltpu.sync_copy(
                x_hbm.at[pl.ds(base, base_blocks * _BLK)],
                xin.at[pl.ds(0, base_blocks * _BLK)])

        pltpu.sync_copy(a_hbm, atab)
        pltpu.sync_copy(b_hbm, btab)

        iota = lax.iota(jnp.int32, _LANES)
        ib8 = [iota + (16 * j % nchan) for j in range(8)]

        def group(g, _):
            offs = g * _BLK
            z = []
            for u in range(_U):
                xs = (xin[pl.ds(offs + u * _LANES, _LANES)]
                      + _RADIUS) * jnp.float32(1.0 / (2.0 * _RADIUS))
                z.append(xs * jnp.float32(_NCELL))

            def step(i, zs):
                zs = list(zs)
                for _ in range(_STEP_UNROLL):
                    out = []
                    for u in range(_U):
                        zu = zs[u]
                        cell = jnp.minimum(
                            jnp.maximum(zu, jnp.float32(0.0)),
                            jnp.float32(_NCELL - 1)).astype(jnp.int32)
                        idx = cell * nchan + ib8[u % 8]
                        ac = plsc.load_gather(atab, [idx])
                        bc = plsc.load_gather(btab, [idx])
                        out.append(ac * zu + bc)
                    zs = out
                return tuple(zs)

            zf = lax.fori_loop(0, _NSTEPS // _STEP_UNROLL, step, tuple(z))

            scale = jnp.float32(2.0 * _RADIUS / _NCELL)
            for u in range(_U):
                xv = xin[pl.ds(offs + u * _LANES, _LANES)]
                xs = (xv + _RADIUS) * jnp.float32(1.0 / (2.0 * _RADIUS))
                msk = jnp.logical_or(xs >= 1.0, xs <= 0.0)
                res = zf[u] * scale - jnp.float32(_RADIUS)
                xout[pl.ds(offs + u * _LANES, _LANES)] = jnp.where(msk, xv, res)
            return 0

        lax.fori_loop(0, myblocks, group, 0)

        @pl.when(is_big)
        def _():
            pltpu.sync_copy(xout, out_hbm.at[pl.ds(base, buf_words)])

        @pl.when(jnp.logical_not(is_big))
        def _():
            pltpu.sync_copy(
                xout.at[pl.ds(0, base_blocks * _BLK)],
                out_hbm.at[pl.ds(base, base_blocks * _BLK)])

    return run(xflat, a1flat, b16flat)


def kernel(x, edge_index, edge_attr, batch, time, theta, B):
    n, nchan = x.shape
    ba = B[0::2, :]
    bb = B[1::2, :]
    a1, b16, s0e = _prep_tables(theta, ba, bb, time)
    y_sc = _sc_transform(x[_TC_ROWS:].reshape(-1),
                         a1.reshape(-1), b16.reshape(-1), nchan)
    y_tc = _tc_transform(x[:_TC_ROWS], s0e)
    out = jnp.concatenate([y_tc, y_sc.reshape(n - _TC_ROWS, nchan)], axis=0)
    return (out, theta)
